# baseline (speedup 1.0000x reference)
import functools

import jax
import jax.numpy as jnp
from jax import lax
from jax.experimental import pallas as pl
from jax.experimental.pallas import tpu as pltpu
from jax.experimental.pallas import tpu_sc as plsc

BATCH = 16384
EMBED = 32
CONTENT = 64
HIDDEN = 64
ROWS_PER_PAGE = 4
PAGES = 250000

NC = 2
NS = 16
NW = NC * NS
B_PER_W = BATCH // NW
CHUNK = 128
NCH = B_PER_W // CHUNK
NG = CHUNK // 16


@functools.lru_cache(maxsize=None)
def _make_sc_gather():
    mesh = plsc.VectorSubcoreMesh(core_axis_name="c", subcore_axis_name="s")

    @functools.partial(
        pl.kernel,
        mesh=mesh,
        compiler_params=pltpu.CompilerParams(use_tc_tiling_on_sc=False,
                                             needs_layout_passes=False),
        out_type=[
            jax.ShapeDtypeStruct((BATCH, EMBED), jnp.float32),
            jax.ShapeDtypeStruct((BATCH, EMBED), jnp.float32),
        ],
        scratch_types=[
            pltpu.VMEM((B_PER_W,), jnp.int32),
            pltpu.VMEM((B_PER_W,), jnp.int32),
            pltpu.VMEM((B_PER_W,), jnp.int32),
            pltpu.VMEM((B_PER_W,), jnp.int32),
            pltpu.VMEM((CHUNK, 128), jnp.float32),
            pltpu.VMEM((CHUNK, 128), jnp.float32),
            pltpu.VMEM((B_PER_W, EMBED), jnp.float32),
            pltpu.VMEM((B_PER_W, EMBED), jnp.float32),
            pltpu.SemaphoreType.DMA,
            pltpu.SemaphoreType.DMA,
        ],
    )
    def _sc_gather(urow_hbm, irow_hbm, uoff_hbm, ioff_hbm, ut_hbm, it_hbm,
                   uout_hbm, iout_hbm,
                   urow_v, irow_v, uoff_v, ioff_v, ug_v, ig_v, uo_v, io_v,
                   usem, isem):
        wid = lax.axis_index("s") * NC + lax.axis_index("c")
        base = wid * B_PER_W
        pltpu.sync_copy(urow_hbm.at[pl.ds(base, B_PER_W)], urow_v)
        pltpu.sync_copy(irow_hbm.at[pl.ds(base, B_PER_W)], irow_v)
        pltpu.sync_copy(uoff_hbm.at[pl.ds(base, B_PER_W)], uoff_v)
        pltpu.sync_copy(ioff_hbm.at[pl.ds(base, B_PER_W)], ioff_v)
        lane = lax.iota(jnp.int32, 16)

        def select(g_v, off_v, o_v, ch):
            for g8 in range(NG):
                loc = lane + (g8 * 16)
                orow = loc + ch * CHUNK
                off = off_v[pl.ds(ch * CHUNK + g8 * 16, 16)]
                for c in range(EMBED):
                    v = plsc.load_gather(g_v, [loc, off + c])
                    plsc.store_scatter(o_v, [orow, jnp.full((16,), c, jnp.int32)], v)

        def body(ch, carry):
            sl = pl.ds(ch * CHUNK, CHUNK)
            ucp = pltpu.async_copy(ut_hbm.at[urow_v.at[sl]], ug_v, usem)
            icp = pltpu.async_copy(it_hbm.at[irow_v.at[sl]], ig_v, isem)
            ucp.wait()
            select(ug_v, uoff_v, uo_v, ch)
            icp.wait()
            select(ig_v, ioff_v, io_v, ch)
            return carry

        lax.fori_loop(0, NCH, body, 0)
        pltpu.sync_copy(uo_v, uout_hbm.at[pl.ds(base, B_PER_W)])
        pltpu.sync_copy(io_v, iout_hbm.at[pl.ds(base, B_PER_W)])

    return _sc_gather


BT = 2048


def _mlp_body(u_ref, i_ref, c_ref, w1u_ref, w1i_ref, w1c_ref, b1_ref,
              w2_ref, b2_ref, o_ref):
    h = (jnp.dot(u_ref[...], w1u_ref[...], preferred_element_type=jnp.float32)
         + jnp.dot(i_ref[...], w1i_ref[...], preferred_element_type=jnp.float32)
         + jnp.dot(c_ref[...], w1c_ref[...], preferred_element_type=jnp.float32)
         + b1_ref[...])
    h = jnp.maximum(h, 0.0)
    o_ref[...] = jnp.sum(h * w2_ref[...], axis=1, keepdims=True) + b2_ref[...]


def _mlp(u_emb, i_emb, content, w1u, w1i, w1c, b1, w2, b2):
    grid = (BATCH // BT,)
    return pl.pallas_call(
        _mlp_body,
        grid=grid,
        in_specs=[
            pl.BlockSpec((BT, EMBED), lambda i: (i, 0)),
            pl.BlockSpec((BT, EMBED), lambda i: (i, 0)),
            pl.BlockSpec((BT, CONTENT), lambda i: (i, 0)),
            pl.BlockSpec((EMBED, HIDDEN), lambda i: (0, 0)),
            pl.BlockSpec((EMBED, HIDDEN), lambda i: (0, 0)),
            pl.BlockSpec((CONTENT, HIDDEN), lambda i: (0, 0)),
            pl.BlockSpec((1, HIDDEN), lambda i: (0, 0)),
            pl.BlockSpec((1, HIDDEN), lambda i: (0, 0)),
            pl.BlockSpec((1, 1), lambda i: (0, 0)),
        ],
        out_specs=pl.BlockSpec((BT, 1), lambda i: (i, 0)),
        out_shape=jax.ShapeDtypeStruct((BATCH, 1), jnp.float32),
    )(u_emb, i_emb, content, w1u, w1i, w1c, b1, w2, b2)


def kernel(user_ids, item_ids, content_features, user_table, item_table,
           W1, b1, W2, b2):
    uids = user_ids.astype(jnp.int32)
    iids = item_ids.astype(jnp.int32)
    urow = uids // ROWS_PER_PAGE
    irow = iids // ROWS_PER_PAGE
    uoff = (uids % ROWS_PER_PAGE) * EMBED
    ioff = (iids % ROWS_PER_PAGE) * EMBED
    ut = user_table.reshape(PAGES, 128)
    it = item_table.reshape(PAGES, 128)
    u_emb, i_emb = _make_sc_gather()(urow, irow, uoff, ioff, ut, it)
    w1u = W1[:EMBED]
    w1i = W1[EMBED:2 * EMBED]
    w1c = W1[2 * EMBED:]
    out = _mlp(u_emb, i_emb, content_features,
               w1u, w1i, w1c,
               b1.reshape(1, HIDDEN),
               W2.reshape(1, HIDDEN),
               b2.reshape(1, 1))
    return out


# speedup vs baseline: 1.7456x; 1.7456x over previous
import functools

import jax
import jax.numpy as jnp
from jax import lax
from jax.experimental import pallas as pl
from jax.experimental.pallas import tpu as pltpu
from jax.experimental.pallas import tpu_sc as plsc

BATCH = 16384
EMBED = 32
CONTENT = 64
HIDDEN = 64
NUSERS = 1000000
ROWS_PER_PAGE = 4

NC = 2
NS = 16
NW = NC * NS
B_PER_W = BATCH // NW
CHUNK = 128
NCH = B_PER_W // CHUNK


TP = 4096
NRUN = -(-NUSERS // TP)
TSTEPS = -(-NRUN // 4)
NPAGE = TSTEPS * TP


def _retile_body(t0_ref, t1_ref, t2_ref, t3_ref, o_ref):
    eye = jnp.eye(EMBED, dtype=jnp.float32)
    for q, t_ref in enumerate((t0_ref, t1_ref, t2_ref, t3_ref)):
        o_ref[:, q * EMBED:(q + 1) * EMBED] = lax.dot_general(
            t_ref[...], eye, (((0,), (0,)), ((), ())),
            preferred_element_type=jnp.float32)


def _retile(table_t):
    def spec(q):
        return pl.BlockSpec(
            (EMBED, TP),
            lambda s, q=q: (0, jnp.minimum(4 * s + q, NRUN - 1)))
    return pl.pallas_call(
        _retile_body,
        grid=(TSTEPS,),
        in_specs=[spec(0), spec(1), spec(2), spec(3)],
        out_specs=pl.BlockSpec((TP, 128), lambda s: (s, 0)),
        out_shape=jax.ShapeDtypeStruct((NPAGE, 128), jnp.float32),
    )(table_t, table_t, table_t, table_t)


@functools.lru_cache(maxsize=None)
def _make_sc_gather():
    mesh = plsc.VectorSubcoreMesh(core_axis_name="c", subcore_axis_name="s")

    @functools.partial(
        pl.kernel,
        mesh=mesh,
        out_type=jax.ShapeDtypeStruct((BATCH, 128), jnp.float32),
        scratch_types=[
            pltpu.VMEM((B_PER_W,), jnp.int32),
            pltpu.VMEM((B_PER_W, 128), jnp.float32),
            pltpu.SemaphoreType.DMA,
        ],
    )
    def _sc_gather(row_hbm, t_hbm, out_hbm, idx_v, rows_v, sem):
        wid = lax.axis_index("s") * NC + lax.axis_index("c")
        base = wid * B_PER_W
        pltpu.sync_copy(row_hbm.at[pl.ds(base, B_PER_W)], idx_v)
        copies = []
        for t in range(NCH):
            sl = pl.ds(t * CHUNK, CHUNK)
            copies.append(pltpu.async_copy(t_hbm.at[idx_v.at[sl]], rows_v.at[sl], sem))
        for cp in copies:
            cp.wait()
        pltpu.sync_copy(rows_v, out_hbm.at[pl.ds(base, B_PER_W)])

    return _sc_gather


BT = 2048


def _mlp_body(u_ref, i_ref, qu_ref, qi_ref, c_ref, w1u_ref, w1i_ref, w1c_ref,
              b1_ref, w2_ref, b2_ref, o_ref):
    blk = lax.broadcasted_iota(jnp.int32, (BT, 128), 1) // EMBED
    um = jnp.where(blk == qu_ref[...], u_ref[...], 0.0)
    im = jnp.where(blk == qi_ref[...], i_ref[...], 0.0)
    h = (jnp.dot(um, w1u_ref[...], preferred_element_type=jnp.float32)
         + jnp.dot(im, w1i_ref[...], preferred_element_type=jnp.float32)
         + jnp.dot(c_ref[...], w1c_ref[...], preferred_element_type=jnp.float32)
         + b1_ref[...])
    h = jnp.maximum(h, 0.0)
    o_ref[...] = jnp.sum(h * w2_ref[...], axis=1, keepdims=True) + b2_ref[...]


def _mlp(u128, i128, qu, qi, content, w1u4, w1i4, w1c, b1, w2, b2):
    grid = (BATCH // BT,)
    return pl.pallas_call(
        _mlp_body,
        grid=grid,
        in_specs=[
            pl.BlockSpec((BT, 128), lambda i: (i, 0)),
            pl.BlockSpec((BT, 128), lambda i: (i, 0)),
            pl.BlockSpec((BT, 1), lambda i: (i, 0)),
            pl.BlockSpec((BT, 1), lambda i: (i, 0)),
            pl.BlockSpec((BT, CONTENT), lambda i: (i, 0)),
            pl.BlockSpec((128, HIDDEN), lambda i: (0, 0)),
            pl.BlockSpec((128, HIDDEN), lambda i: (0, 0)),
            pl.BlockSpec((CONTENT, HIDDEN), lambda i: (0, 0)),
            pl.BlockSpec((1, HIDDEN), lambda i: (0, 0)),
            pl.BlockSpec((1, HIDDEN), lambda i: (0, 0)),
            pl.BlockSpec((1, 1), lambda i: (0, 0)),
        ],
        out_specs=pl.BlockSpec((BT, 1), lambda i: (i, 0)),
        out_shape=jax.ShapeDtypeStruct((BATCH, 1), jnp.float32),
    )(u128, i128, qu, qi, content, w1u4, w1i4, w1c, b1, w2, b2)


def kernel(user_ids, item_ids, content_features, user_table, item_table,
           W1, b1, W2, b2):
    uids = user_ids.astype(jnp.int32)
    iids = item_ids.astype(jnp.int32)
    ur = uids // TP
    ir = iids // TP
    urow = (ur // 4) * TP + uids % TP
    irow = (ir // 4) * TP + iids % TP
    qu = (ur % 4).reshape(BATCH, 1)
    qi = (ir % 4).reshape(BATCH, 1)
    ut = _retile(user_table.T)
    it = _retile(item_table.T)
    gather = _make_sc_gather()
    u128 = gather(urow, ut)
    i128 = gather(irow, it)
    w1u = W1[:EMBED]
    w1i = W1[EMBED:2 * EMBED]
    w1c = W1[2 * EMBED:]
    w1u4 = jnp.tile(w1u, (ROWS_PER_PAGE, 1))
    w1i4 = jnp.tile(w1i, (ROWS_PER_PAGE, 1))
    out = _mlp(u128, i128, qu, qi, content_features,
               w1u4, w1i4, w1c,
               b1.reshape(1, HIDDEN),
               W2.reshape(1, HIDDEN),
               b2.reshape(1, 1))
    return out


# speedup vs baseline: 1.8128x; 1.0385x over previous
import functools

import jax
import jax.numpy as jnp
from jax import lax
from jax.experimental import pallas as pl
from jax.experimental.pallas import tpu as pltpu
from jax.experimental.pallas import tpu_sc as plsc

BATCH = 16384
EMBED = 32
CONTENT = 64
HIDDEN = 64
NUSERS = 1000000
ROWS_PER_PAGE = 4

NC = 2
NS = 16
NW = NC * NS
B_PER_W = BATCH // NW
CHUNK = 128
NCH = B_PER_W // CHUNK


TP = 4096
NRUN = -(-NUSERS // TP)
TSTEPS = -(-NRUN // 2)
NPAGE = TSTEPS * TP


def _tw_body(t0_ref, t1_ref, w1_ref, o_ref):
    for q, t_ref in enumerate((t0_ref, t1_ref)):
        o_ref[:, q * HIDDEN:(q + 1) * HIDDEN] = lax.dot_general(
            t_ref[...], w1_ref[...], (((0,), (0,)), ((), ())),
            preferred_element_type=jnp.float32)


def _tw(table_t, w1):
    def spec(q):
        return pl.BlockSpec(
            (EMBED, TP),
            lambda s, q=q: (0, jnp.minimum(2 * s + q, NRUN - 1)))
    return pl.pallas_call(
        _tw_body,
        grid=(TSTEPS,),
        in_specs=[spec(0), spec(1),
                  pl.BlockSpec((EMBED, HIDDEN), lambda s: (0, 0))],
        out_specs=pl.BlockSpec((TP, 128), lambda s: (s, 0)),
        out_shape=jax.ShapeDtypeStruct((NPAGE, 128), jnp.float32),
    )(table_t, table_t, w1)


@functools.lru_cache(maxsize=None)
def _make_sc_gather():
    mesh = plsc.VectorSubcoreMesh(core_axis_name="c", subcore_axis_name="s")

    @functools.partial(
        pl.kernel,
        mesh=mesh,
        out_type=jax.ShapeDtypeStruct((BATCH, 128), jnp.float32),
        scratch_types=[
            pltpu.VMEM((B_PER_W,), jnp.int32),
            pltpu.VMEM((B_PER_W, 128), jnp.float32),
            pltpu.SemaphoreType.DMA,
        ],
    )
    def _sc_gather(row_hbm, t_hbm, out_hbm, idx_v, rows_v, sem):
        wid = lax.axis_index("s") * NC + lax.axis_index("c")
        base = wid * B_PER_W
        pltpu.sync_copy(row_hbm.at[pl.ds(base, B_PER_W)], idx_v)
        copies = []
        for t in range(NCH):
            sl = pl.ds(t * CHUNK, CHUNK)
            copies.append(pltpu.async_copy(t_hbm.at[idx_v.at[sl]], rows_v.at[sl], sem))
        for cp in copies:
            cp.wait()
        pltpu.sync_copy(rows_v, out_hbm.at[pl.ds(base, B_PER_W)])

    return _sc_gather


BT = 2048


def _mlp_body(u_ref, i_ref, qu_ref, qi_ref, c_ref, w1c_ref,
              b1_ref, w2_ref, b2_ref, o_ref):
    grp = lax.broadcasted_iota(jnp.int32, (BT, 128), 1) // HIDDEN
    su = jnp.where(grp == qu_ref[...], u_ref[...], 0.0)
    si = jnp.where(grp == qi_ref[...], i_ref[...], 0.0)
    tw = (su[:, :HIDDEN] + su[:, HIDDEN:]) + (si[:, :HIDDEN] + si[:, HIDDEN:])
    h = (tw
         + jnp.dot(c_ref[...], w1c_ref[...], preferred_element_type=jnp.float32)
         + b1_ref[...])
    h = jnp.maximum(h, 0.0)
    o_ref[...] = jnp.sum(h * w2_ref[...], axis=1, keepdims=True) + b2_ref[...]


def _mlp(u128, i128, qu, qi, content, w1c, b1, w2, b2):
    grid = (BATCH // BT,)
    return pl.pallas_call(
        _mlp_body,
        grid=grid,
        in_specs=[
            pl.BlockSpec((BT, 128), lambda i: (i, 0)),
            pl.BlockSpec((BT, 128), lambda i: (i, 0)),
            pl.BlockSpec((BT, 1), lambda i: (i, 0)),
            pl.BlockSpec((BT, 1), lambda i: (i, 0)),
            pl.BlockSpec((BT, CONTENT), lambda i: (i, 0)),
            pl.BlockSpec((CONTENT, HIDDEN), lambda i: (0, 0)),
            pl.BlockSpec((1, HIDDEN), lambda i: (0, 0)),
            pl.BlockSpec((1, HIDDEN), lambda i: (0, 0)),
            pl.BlockSpec((1, 1), lambda i: (0, 0)),
        ],
        out_specs=pl.BlockSpec((BT, 1), lambda i: (i, 0)),
        out_shape=jax.ShapeDtypeStruct((BATCH, 1), jnp.float32),
    )(u128, i128, qu, qi, content, w1c, b1, w2, b2)


def kernel(user_ids, item_ids, content_features, user_table, item_table,
           W1, b1, W2, b2):
    uids = user_ids.astype(jnp.int32)
    iids = item_ids.astype(jnp.int32)
    ur = uids // TP
    ir = iids // TP
    urow = (ur // 2) * TP + uids % TP
    irow = (ir // 2) * TP + iids % TP
    qu = (ur % 2).reshape(BATCH, 1)
    qi = (ir % 2).reshape(BATCH, 1)
    w1u = W1[:EMBED]
    w1i = W1[EMBED:2 * EMBED]
    w1c = W1[2 * EMBED:]
    ut = _tw(user_table.T, w1u)
    it = _tw(item_table.T, w1i)
    gather = _make_sc_gather()
    u128 = gather(urow, ut)
    i128 = gather(irow, it)
    out = _mlp(u128, i128, qu, qi, content_features,
               w1c,
               b1.reshape(1, HIDDEN),
               W2.reshape(1, HIDDEN),
               b2.reshape(1, 1))
    return out


# speedup vs baseline: 3.6698x; 2.0243x over previous
import functools

import jax
import jax.numpy as jnp
from jax import lax
from jax.experimental import pallas as pl
from jax.experimental.pallas import tpu as pltpu
from jax.experimental.pallas import tpu_sc as plsc

BATCH = 16384
EMBED = 32
CONTENT = 64
HIDDEN = 64
NUSERS = 1000000
ROWS_PER_PAGE = 4

NC = 2
NS = 16
NW = NC * NS
B_PER_W = BATCH // NW
CHUNK = 128
NCH = B_PER_W // CHUNK


TP = 4096
NRUN = -(-NUSERS // TP)
TSTEPS = -(-NRUN // 4)
NPAGE = TSTEPS * TP


def _retile_body(t0_ref, t1_ref, t2_ref, t3_ref, eye_ref, o_ref):
    x = jnp.concatenate(
        [t0_ref[...], t1_ref[...], t2_ref[...], t3_ref[...]], axis=0)
    o_ref[...] = lax.dot_general(
        x, eye_ref[...], (((0,), (0,)), ((), ())),
        preferred_element_type=jnp.float32)


def _retile(table_t, eye):
    def spec(q):
        return pl.BlockSpec(
            (EMBED, TP),
            lambda s, q=q: (0, jnp.minimum(4 * s + q, NRUN - 1)))
    return pl.pallas_call(
        _retile_body,
        grid=(TSTEPS,),
        in_specs=[spec(0), spec(1), spec(2), spec(3),
                  pl.BlockSpec((128, 128), lambda s: (0, 0))],
        out_specs=pl.BlockSpec((TP, 128), lambda s: (s, 0)),
        out_shape=jax.ShapeDtypeStruct((NPAGE, 128), jnp.float32),
    )(table_t, table_t, table_t, table_t, eye)


@functools.lru_cache(maxsize=None)
def _make_sc_gather():
    mesh = plsc.VectorSubcoreMesh(core_axis_name="c", subcore_axis_name="s")

    @functools.partial(
        pl.kernel,
        mesh=mesh,
        out_type=jax.ShapeDtypeStruct((BATCH, 128), jnp.float32),
        scratch_types=[
            pltpu.VMEM((B_PER_W,), jnp.int32),
            pltpu.VMEM((B_PER_W, 128), jnp.float32),
            pltpu.SemaphoreType.DMA,
        ],
    )
    def _sc_gather(row_hbm, t_hbm, out_hbm, idx_v, rows_v, sem):
        wid = lax.axis_index("s") * NC + lax.axis_index("c")
        base = wid * B_PER_W
        pltpu.sync_copy(row_hbm.at[pl.ds(base, B_PER_W)], idx_v)
        copies = []
        for t in range(NCH):
            sl = pl.ds(t * CHUNK, CHUNK)
            copies.append(pltpu.async_copy(t_hbm.at[idx_v.at[sl]], rows_v.at[sl], sem))
        for cp in copies:
            cp.wait()
        pltpu.sync_copy(rows_v, out_hbm.at[pl.ds(base, B_PER_W)])

    return _sc_gather


BT = 2048


def _mlp_body(u_ref, i_ref, qu_ref, qi_ref, c_ref, w1u4_ref, w1i4_ref,
              w1c_ref, b1_ref, w2_ref, b2_ref, o_ref):
    grp = lax.broadcasted_iota(jnp.int32, (BT, 128), 1) // EMBED
    um = jnp.where(grp == qu_ref[...], u_ref[...], 0.0)
    im = jnp.where(grp == qi_ref[...], i_ref[...], 0.0)
    h = (jnp.dot(um, w1u4_ref[...], preferred_element_type=jnp.float32)
         + jnp.dot(im, w1i4_ref[...], preferred_element_type=jnp.float32)
         + jnp.dot(c_ref[...], w1c_ref[...], preferred_element_type=jnp.float32)
         + b1_ref[...])
    h = jnp.maximum(h, 0.0)
    o_ref[...] = jnp.sum(h * w2_ref[...], axis=1, keepdims=True) + b2_ref[...]


def _mlp(u128, i128, qu, qi, content, w1u4, w1i4, w1c, b1, w2, b2):
    grid = (BATCH // BT,)
    return pl.pallas_call(
        _mlp_body,
        grid=grid,
        in_specs=[
            pl.BlockSpec((BT, 128), lambda i: (i, 0)),
            pl.BlockSpec((BT, 128), lambda i: (i, 0)),
            pl.BlockSpec((BT, 1), lambda i: (i, 0)),
            pl.BlockSpec((BT, 1), lambda i: (i, 0)),
            pl.BlockSpec((BT, CONTENT), lambda i: (i, 0)),
            pl.BlockSpec((128, HIDDEN), lambda i: (0, 0)),
            pl.BlockSpec((128, HIDDEN), lambda i: (0, 0)),
            pl.BlockSpec((CONTENT, HIDDEN), lambda i: (0, 0)),
            pl.BlockSpec((1, HIDDEN), lambda i: (0, 0)),
            pl.BlockSpec((1, HIDDEN), lambda i: (0, 0)),
            pl.BlockSpec((1, 1), lambda i: (0, 0)),
        ],
        out_specs=pl.BlockSpec((BT, 1), lambda i: (i, 0)),
        out_shape=jax.ShapeDtypeStruct((BATCH, 1), jnp.float32),
    )(u128, i128, qu, qi, content, w1u4, w1i4, w1c, b1, w2, b2)


def kernel(user_ids, item_ids, content_features, user_table, item_table,
           W1, b1, W2, b2):
    uids = user_ids.astype(jnp.int32)
    iids = item_ids.astype(jnp.int32)
    ur = uids // TP
    ir = iids // TP
    urow = (ur // 4) * TP + uids % TP
    irow = (ir // 4) * TP + iids % TP
    qu = (ur % 4).reshape(BATCH, 1)
    qi = (ir % 4).reshape(BATCH, 1)
    w1u = W1[:EMBED]
    w1i = W1[EMBED:2 * EMBED]
    w1c = W1[2 * EMBED:]
    eye = jnp.eye(128, dtype=jnp.float32)
    ut = _retile(user_table.T, eye)
    it = _retile(item_table.T, eye)
    gather = _make_sc_gather()
    u128 = gather(urow, ut)
    i128 = gather(irow, it)
    w1u4 = jnp.tile(w1u, (ROWS_PER_PAGE, 1))
    w1i4 = jnp.tile(w1i, (ROWS_PER_PAGE, 1))
    out = _mlp(u128, i128, qu, qi, content_features,
               w1u4, w1i4, w1c,
               b1.reshape(1, HIDDEN),
               W2.reshape(1, HIDDEN),
               b2.reshape(1, 1))
    return out


# speedup vs baseline: 4.1320x; 1.1260x over previous
import functools

import jax
import jax.numpy as jnp
from jax import lax
from jax.experimental import pallas as pl
from jax.experimental.pallas import tpu as pltpu
from jax.experimental.pallas import tpu_sc as plsc

BATCH = 16384
EMBED = 32
CONTENT = 64
HIDDEN = 64
NUSERS = 1000000
ROWS_PER_PAGE = 4

NC = 2
NS = 16
NW = NC * NS
B_PER_W = BATCH // NW
CHUNK = 128
NCH = B_PER_W // CHUNK


TP = 8192
NRUN = -(-NUSERS // TP)
TSTEPS = -(-NRUN // 4)
NPAGE = TSTEPS * TP


def _retile_body(t0_ref, t1_ref, t2_ref, t3_ref, eye_ref, o_ref):
    x = jnp.concatenate(
        [t0_ref[...], t1_ref[...], t2_ref[...], t3_ref[...]], axis=0)
    o_ref[...] = lax.dot_general(
        x, eye_ref[...], (((0,), (0,)), ((), ())),
        preferred_element_type=jnp.float32)


def _retile(table_t, eye):
    def spec(q):
        return pl.BlockSpec(
            (EMBED, TP),
            lambda s, q=q: (0, jnp.minimum(4 * s + q, NRUN - 1)))
    return pl.pallas_call(
        _retile_body,
        grid=(TSTEPS,),
        in_specs=[spec(0), spec(1), spec(2), spec(3),
                  pl.BlockSpec((128, 128), lambda s: (0, 0))],
        out_specs=pl.BlockSpec((TP, 128), lambda s: (s, 0)),
        out_shape=jax.ShapeDtypeStruct((NPAGE, 128), jnp.float32),
    )(table_t, table_t, table_t, table_t, eye)


@functools.lru_cache(maxsize=None)
def _make_sc_gather():
    mesh = plsc.VectorSubcoreMesh(core_axis_name="c", subcore_axis_name="s")

    @functools.partial(
        pl.kernel,
        mesh=mesh,
        out_type=jax.ShapeDtypeStruct((BATCH, 128), jnp.float32),
        scratch_types=[
            pltpu.VMEM((B_PER_W,), jnp.int32),
            pltpu.VMEM((B_PER_W, 128), jnp.float32),
            pltpu.SemaphoreType.DMA,
        ],
    )
    def _sc_gather(row_hbm, t_hbm, out_hbm, idx_v, rows_v, sem):
        wid = lax.axis_index("s") * NC + lax.axis_index("c")
        base = wid * B_PER_W
        pltpu.sync_copy(row_hbm.at[pl.ds(base, B_PER_W)], idx_v)
        copies = []
        for t in range(NCH):
            sl = pl.ds(t * CHUNK, CHUNK)
            copies.append(pltpu.async_copy(t_hbm.at[idx_v.at[sl]], rows_v.at[sl], sem))
        for cp in copies:
            cp.wait()
        pltpu.sync_copy(rows_v, out_hbm.at[pl.ds(base, B_PER_W)])

    return _sc_gather


BT = 2048


def _mlp_body(u_ref, i_ref, qu_ref, qi_ref, c_ref, w1u4_ref, w1i4_ref,
              w1c_ref, b1_ref, w2_ref, b2_ref, o_ref):
    grp = lax.broadcasted_iota(jnp.int32, (BT, 128), 1) // EMBED
    um = jnp.where(grp == qu_ref[...], u_ref[...], 0.0)
    im = jnp.where(grp == qi_ref[...], i_ref[...], 0.0)
    h = (jnp.dot(um, w1u4_ref[...], preferred_element_type=jnp.float32)
         + jnp.dot(im, w1i4_ref[...], preferred_element_type=jnp.float32)
         + jnp.dot(c_ref[...], w1c_ref[...], preferred_element_type=jnp.float32)
         + b1_ref[...])
    h = jnp.maximum(h, 0.0)
    o_ref[...] = jnp.sum(h * w2_ref[...], axis=1, keepdims=True) + b2_ref[...]


def _mlp(u128, i128, qu, qi, content, w1u4, w1i4, w1c, b1, w2, b2):
    grid = (BATCH // BT,)
    return pl.pallas_call(
        _mlp_body,
        grid=grid,
        in_specs=[
            pl.BlockSpec((BT, 128), lambda i: (i, 0)),
            pl.BlockSpec((BT, 128), lambda i: (i, 0)),
            pl.BlockSpec((BT, 1), lambda i: (i, 0)),
            pl.BlockSpec((BT, 1), lambda i: (i, 0)),
            pl.BlockSpec((BT, CONTENT), lambda i: (i, 0)),
            pl.BlockSpec((128, HIDDEN), lambda i: (0, 0)),
            pl.BlockSpec((128, HIDDEN), lambda i: (0, 0)),
            pl.BlockSpec((CONTENT, HIDDEN), lambda i: (0, 0)),
            pl.BlockSpec((1, HIDDEN), lambda i: (0, 0)),
            pl.BlockSpec((1, HIDDEN), lambda i: (0, 0)),
            pl.BlockSpec((1, 1), lambda i: (0, 0)),
        ],
        out_specs=pl.BlockSpec((BT, 1), lambda i: (i, 0)),
        out_shape=jax.ShapeDtypeStruct((BATCH, 1), jnp.float32),
    )(u128, i128, qu, qi, content, w1u4, w1i4, w1c, b1, w2, b2)


def kernel(user_ids, item_ids, content_features, user_table, item_table,
           W1, b1, W2, b2):
    uids = user_ids.astype(jnp.int32)
    iids = item_ids.astype(jnp.int32)
    ur = uids // TP
    ir = iids // TP
    urow = (ur // 4) * TP + uids % TP
    irow = (ir // 4) * TP + iids % TP
    qu = (ur % 4).reshape(BATCH, 1)
    qi = (ir % 4).reshape(BATCH, 1)
    w1u = W1[:EMBED]
    w1i = W1[EMBED:2 * EMBED]
    w1c = W1[2 * EMBED:]
    eye = jnp.eye(128, dtype=jnp.float32)
    ut = _retile(user_table.T, eye)
    it = _retile(item_table.T, eye)
    gather = _make_sc_gather()
    u128 = gather(urow, ut)
    i128 = gather(irow, it)
    w1u4 = jnp.tile(w1u, (ROWS_PER_PAGE, 1))
    w1i4 = jnp.tile(w1i, (ROWS_PER_PAGE, 1))
    out = _mlp(u128, i128, qu, qi, content_features,
               w1u4, w1i4, w1c,
               b1.reshape(1, HIDDEN),
               W2.reshape(1, HIDDEN),
               b2.reshape(1, 1))
    return out


# speedup vs baseline: 4.1676x; 1.0086x over previous
import functools

import jax
import jax.numpy as jnp
from jax import lax
from jax.experimental import pallas as pl
from jax.experimental.pallas import tpu as pltpu
from jax.experimental.pallas import tpu_sc as plsc

BATCH = 16384
EMBED = 32
CONTENT = 64
HIDDEN = 64
NUSERS = 1000000
ROWS_PER_PAGE = 4

NC = 2
NS = 16
NW = NC * NS
B_PER_W = BATCH // NW
CHUNK = 128
NCH = B_PER_W // CHUNK


TP = 16384
NRUN = -(-NUSERS // TP)
TSTEPS = -(-NRUN // 4)
NPAGE = TSTEPS * TP


def _retile_body(t0_ref, t1_ref, t2_ref, t3_ref, eye_ref, o_ref):
    x = jnp.concatenate(
        [t0_ref[...], t1_ref[...], t2_ref[...], t3_ref[...]], axis=0)
    o_ref[...] = lax.dot_general(
        x, eye_ref[...], (((0,), (0,)), ((), ())),
        preferred_element_type=jnp.float32)


def _retile(table_t, eye):
    def spec(q):
        return pl.BlockSpec(
            (EMBED, TP),
            lambda s, q=q: (0, jnp.minimum(4 * s + q, NRUN - 1)))
    return pl.pallas_call(
        _retile_body,
        grid=(TSTEPS,),
        in_specs=[spec(0), spec(1), spec(2), spec(3),
                  pl.BlockSpec((128, 128), lambda s: (0, 0))],
        out_specs=pl.BlockSpec((TP, 128), lambda s: (s, 0)),
        out_shape=jax.ShapeDtypeStruct((NPAGE, 128), jnp.float32),
    )(table_t, table_t, table_t, table_t, eye)


@functools.lru_cache(maxsize=None)
def _make_sc_gather():
    mesh = plsc.VectorSubcoreMesh(core_axis_name="c", subcore_axis_name="s")

    @functools.partial(
        pl.kernel,
        mesh=mesh,
        out_type=jax.ShapeDtypeStruct((BATCH, 128), jnp.float32),
        scratch_types=[
            pltpu.VMEM((B_PER_W,), jnp.int32),
            pltpu.VMEM((B_PER_W, 128), jnp.float32),
            pltpu.SemaphoreType.DMA,
        ],
    )
    def _sc_gather(row_hbm, t_hbm, out_hbm, idx_v, rows_v, sem):
        wid = lax.axis_index("s") * NC + lax.axis_index("c")
        base = wid * B_PER_W
        pltpu.sync_copy(row_hbm.at[pl.ds(base, B_PER_W)], idx_v)
        copies = []
        for t in range(NCH):
            sl = pl.ds(t * CHUNK, CHUNK)
            copies.append(pltpu.async_copy(t_hbm.at[idx_v.at[sl]], rows_v.at[sl], sem))
        for cp in copies:
            cp.wait()
        pltpu.sync_copy(rows_v, out_hbm.at[pl.ds(base, B_PER_W)])

    return _sc_gather


BT = 2048


def _mlp_body(u_ref, i_ref, qu_ref, qi_ref, c_ref, w1u4_ref, w1i4_ref,
              w1c_ref, b1_ref, w2_ref, b2_ref, o_ref):
    grp = lax.broadcasted_iota(jnp.int32, (BT, 128), 1) // EMBED
    um = jnp.where(grp == qu_ref[...], u_ref[...], 0.0)
    im = jnp.where(grp == qi_ref[...], i_ref[...], 0.0)
    h = (jnp.dot(um, w1u4_ref[...], preferred_element_type=jnp.float32)
         + jnp.dot(im, w1i4_ref[...], preferred_element_type=jnp.float32)
         + jnp.dot(c_ref[...], w1c_ref[...], preferred_element_type=jnp.float32)
         + b1_ref[...])
    h = jnp.maximum(h, 0.0)
    o_ref[...] = jnp.sum(h * w2_ref[...], axis=1, keepdims=True) + b2_ref[...]


def _mlp(u128, i128, qu, qi, content, w1u4, w1i4, w1c, b1, w2, b2):
    grid = (BATCH // BT,)
    return pl.pallas_call(
        _mlp_body,
        grid=grid,
        in_specs=[
            pl.BlockSpec((BT, 128), lambda i: (i, 0)),
            pl.BlockSpec((BT, 128), lambda i: (i, 0)),
            pl.BlockSpec((BT, 1), lambda i: (i, 0)),
            pl.BlockSpec((BT, 1), lambda i: (i, 0)),
            pl.BlockSpec((BT, CONTENT), lambda i: (i, 0)),
            pl.BlockSpec((128, HIDDEN), lambda i: (0, 0)),
            pl.BlockSpec((128, HIDDEN), lambda i: (0, 0)),
            pl.BlockSpec((CONTENT, HIDDEN), lambda i: (0, 0)),
            pl.BlockSpec((1, HIDDEN), lambda i: (0, 0)),
            pl.BlockSpec((1, HIDDEN), lambda i: (0, 0)),
            pl.BlockSpec((1, 1), lambda i: (0, 0)),
        ],
        out_specs=pl.BlockSpec((BT, 1), lambda i: (i, 0)),
        out_shape=jax.ShapeDtypeStruct((BATCH, 1), jnp.float32),
    )(u128, i128, qu, qi, content, w1u4, w1i4, w1c, b1, w2, b2)


def kernel(user_ids, item_ids, content_features, user_table, item_table,
           W1, b1, W2, b2):
    uids = user_ids.astype(jnp.int32)
    iids = item_ids.astype(jnp.int32)
    ur = uids // TP
    ir = iids // TP
    urow = (ur // 4) * TP + uids % TP
    irow = (ir // 4) * TP + iids % TP
    qu = (ur % 4).reshape(BATCH, 1)
    qi = (ir % 4).reshape(BATCH, 1)
    w1u = W1[:EMBED]
    w1i = W1[EMBED:2 * EMBED]
    w1c = W1[2 * EMBED:]
    eye = jnp.eye(128, dtype=jnp.float32)
    ut = _retile(user_table.T, eye)
    it = _retile(item_table.T, eye)
    gather = _make_sc_gather()
    u128 = gather(urow, ut)
    i128 = gather(irow, it)
    w1u4 = jnp.tile(w1u, (ROWS_PER_PAGE, 1))
    w1i4 = jnp.tile(w1i, (ROWS_PER_PAGE, 1))
    out = _mlp(u128, i128, qu, qi, content_features,
               w1u4, w1i4, w1c,
               b1.reshape(1, HIDDEN),
               W2.reshape(1, HIDDEN),
               b2.reshape(1, 1))
    return out


# speedup vs baseline: 4.5426x; 1.0900x over previous
import functools

import jax
import jax.numpy as jnp
from jax import lax
from jax.experimental import pallas as pl
from jax.experimental.pallas import tpu as pltpu
from jax.experimental.pallas import tpu_sc as plsc

BATCH = 16384
EMBED = 32
CONTENT = 64
HIDDEN = 64
NUSERS = 1000000
ROWS_PER_PAGE = 4

NC = 2
NS = 16
NW = NC * NS
B_PER_W = BATCH // NW
CHUNK = 128
NCH = B_PER_W // CHUNK


TP = 8192
NRUN = -(-NUSERS // TP)
NJQ = 8
TSTEPS = -(-NRUN // NJQ)
NPAGE = TSTEPS * TP


def _retile_body(*refs):
    t_refs, eye_ref, o_ref = refs[:NJQ], refs[NJQ], refs[NJQ + 1]
    halves = []
    for j in range(2):
        x = jnp.concatenate([t_refs[4 * j + q][...] for q in range(4)], axis=0)
        y = lax.dot_general(x, eye_ref[...], (((0,), (0,)), ((), ())),
                            preferred_element_type=jnp.float32)
        u16 = lax.bitcast_convert_type(y.astype(jnp.bfloat16), jnp.uint16)
        halves.append(u16.astype(jnp.uint32))
    o_ref[...] = halves[0] | (halves[1] << 16)


def _retile(table_t, eye):
    def spec(rq):
        return pl.BlockSpec(
            (EMBED, TP),
            lambda s, rq=rq: (0, jnp.minimum(NJQ * s + rq, NRUN - 1)))
    return pl.pallas_call(
        _retile_body,
        grid=(TSTEPS,),
        in_specs=[spec(rq) for rq in range(NJQ)]
        + [pl.BlockSpec((128, 128), lambda s: (0, 0))],
        out_specs=pl.BlockSpec((TP, 128), lambda s: (s, 0)),
        out_shape=jax.ShapeDtypeStruct((NPAGE, 128), jnp.uint32),
    )(*([table_t] * NJQ), eye)


@functools.lru_cache(maxsize=None)
def _make_sc_gather():
    mesh = plsc.VectorSubcoreMesh(core_axis_name="c", subcore_axis_name="s")

    @functools.partial(
        pl.kernel,
        mesh=mesh,
        out_type=jax.ShapeDtypeStruct((BATCH, 128), jnp.uint32),
        scratch_types=[
            pltpu.VMEM((B_PER_W,), jnp.int32),
            pltpu.VMEM((B_PER_W, 128), jnp.uint32),
            pltpu.SemaphoreType.DMA,
        ],
    )
    def _sc_gather(row_hbm, t_hbm, out_hbm, idx_v, rows_v, sem):
        wid = lax.axis_index("s") * NC + lax.axis_index("c")
        base = wid * B_PER_W
        pltpu.sync_copy(row_hbm.at[pl.ds(base, B_PER_W)], idx_v)
        copies = []
        for t in range(NCH):
            sl = pl.ds(t * CHUNK, CHUNK)
            copies.append(pltpu.async_copy(t_hbm.at[idx_v.at[sl]], rows_v.at[sl], sem))
        for cp in copies:
            cp.wait()
        pltpu.sync_copy(rows_v, out_hbm.at[pl.ds(base, B_PER_W)])

    return _sc_gather


BT = 2048


def _mlp_body(u_ref, i_ref, qu_ref, qi_ref, ju_ref, ji_ref, c_ref,
              w1u4_ref, w1i4_ref, w1c_ref, b1_ref, w2_ref, b2_ref, o_ref):
    grp = lax.broadcasted_iota(jnp.int32, (BT, 128), 1) // EMBED

    def unpack(w_ref, j_ref):
        w = w_ref[...]
        bits = jnp.where(j_ref[...] == 0, w << 16, w & jnp.uint32(0xFFFF0000))
        return lax.bitcast_convert_type(bits, jnp.float32)

    um = jnp.where(grp == qu_ref[...], unpack(u_ref, ju_ref), 0.0)
    im = jnp.where(grp == qi_ref[...], unpack(i_ref, ji_ref), 0.0)
    h = (jnp.dot(um, w1u4_ref[...], preferred_element_type=jnp.float32)
         + jnp.dot(im, w1i4_ref[...], preferred_element_type=jnp.float32)
         + jnp.dot(c_ref[...], w1c_ref[...], preferred_element_type=jnp.float32)
         + b1_ref[...])
    h = jnp.maximum(h, 0.0)
    o_ref[...] = jnp.sum(h * w2_ref[...], axis=1, keepdims=True) + b2_ref[...]


def _mlp(u128, i128, qu, qi, ju, ji, content, w1u4, w1i4, w1c, b1, w2, b2):
    grid = (BATCH // BT,)
    return pl.pallas_call(
        _mlp_body,
        grid=grid,
        in_specs=[
            pl.BlockSpec((BT, 128), lambda i: (i, 0)),
            pl.BlockSpec((BT, 128), lambda i: (i, 0)),
            pl.BlockSpec((BT, 1), lambda i: (i, 0)),
            pl.BlockSpec((BT, 1), lambda i: (i, 0)),
            pl.BlockSpec((BT, 1), lambda i: (i, 0)),
            pl.BlockSpec((BT, 1), lambda i: (i, 0)),
            pl.BlockSpec((BT, CONTENT), lambda i: (i, 0)),
            pl.BlockSpec((128, HIDDEN), lambda i: (0, 0)),
            pl.BlockSpec((128, HIDDEN), lambda i: (0, 0)),
            pl.BlockSpec((CONTENT, HIDDEN), lambda i: (0, 0)),
            pl.BlockSpec((1, HIDDEN), lambda i: (0, 0)),
            pl.BlockSpec((1, HIDDEN), lambda i: (0, 0)),
            pl.BlockSpec((1, 1), lambda i: (0, 0)),
        ],
        out_specs=pl.BlockSpec((BT, 1), lambda i: (i, 0)),
        out_shape=jax.ShapeDtypeStruct((BATCH, 1), jnp.float32),
    )(u128, i128, qu, qi, ju, ji, content, w1u4, w1i4, w1c, b1, w2, b2)


def kernel(user_ids, item_ids, content_features, user_table, item_table,
           W1, b1, W2, b2):
    uids = user_ids.astype(jnp.int32)
    iids = item_ids.astype(jnp.int32)
    ur = uids // TP
    ir = iids // TP
    urow = (ur // NJQ) * TP + uids % TP
    irow = (ir // NJQ) * TP + iids % TP
    qu = (ur % 4).reshape(BATCH, 1)
    qi = (ir % 4).reshape(BATCH, 1)
    ju = ((ur // 4) % 2).reshape(BATCH, 1)
    ji = ((ir // 4) % 2).reshape(BATCH, 1)
    w1u = W1[:EMBED]
    w1i = W1[EMBED:2 * EMBED]
    w1c = W1[2 * EMBED:]
    eye = jnp.eye(128, dtype=jnp.float32)
    ut = _retile(user_table.T, eye)
    it = _retile(item_table.T, eye)
    gather = _make_sc_gather()
    u128 = gather(urow, ut)
    i128 = gather(irow, it)
    w1u4 = jnp.tile(w1u, (ROWS_PER_PAGE, 1))
    w1i4 = jnp.tile(w1i, (ROWS_PER_PAGE, 1))
    out = _mlp(u128, i128, qu, qi, ju, ji, content_features,
               w1u4, w1i4, w1c,
               b1.reshape(1, HIDDEN),
               W2.reshape(1, HIDDEN),
               b2.reshape(1, 1))
    return out


# speedup vs baseline: 4.5984x; 1.0123x over previous
import functools

import jax
import jax.numpy as jnp
from jax import lax
from jax.experimental import pallas as pl
from jax.experimental.pallas import tpu as pltpu
from jax.experimental.pallas import tpu_sc as plsc

BATCH = 16384
EMBED = 32
CONTENT = 64
HIDDEN = 64
NUSERS = 1000000
ROWS_PER_PAGE = 4

NC = 2
NS = 16
NW = NC * NS
B_PER_W = BATCH // NW
CHUNK = 128
NCH = B_PER_W // CHUNK


TP = 16384
NRUN = -(-NUSERS // TP)
NJQ = 8
TSTEPS = -(-NRUN // NJQ)
NPAGE = TSTEPS * TP


def _retile_body(*refs):
    t_refs, eye_ref, o_ref = refs[:NJQ], refs[NJQ], refs[NJQ + 1]
    halves = []
    for j in range(2):
        x = jnp.concatenate([t_refs[4 * j + q][...] for q in range(4)], axis=0)
        y = lax.dot_general(x, eye_ref[...], (((0,), (0,)), ((), ())),
                            preferred_element_type=jnp.float32)
        u16 = lax.bitcast_convert_type(y.astype(jnp.bfloat16), jnp.uint16)
        halves.append(u16.astype(jnp.uint32))
    o_ref[...] = halves[0] | (halves[1] << 16)


def _retile(table_t, eye):
    def spec(rq):
        return pl.BlockSpec(
            (EMBED, TP),
            lambda s, rq=rq: (0, jnp.minimum(NJQ * s + rq, NRUN - 1)))
    return pl.pallas_call(
        _retile_body,
        grid=(TSTEPS,),
        in_specs=[spec(rq) for rq in range(NJQ)]
        + [pl.BlockSpec((128, 128), lambda s: (0, 0))],
        out_specs=pl.BlockSpec((TP, 128), lambda s: (s, 0)),
        out_shape=jax.ShapeDtypeStruct((NPAGE, 128), jnp.uint32),
    )(*([table_t] * NJQ), eye)


@functools.lru_cache(maxsize=None)
def _make_sc_gather():
    mesh = plsc.VectorSubcoreMesh(core_axis_name="c", subcore_axis_name="s")

    @functools.partial(
        pl.kernel,
        mesh=mesh,
        out_type=jax.ShapeDtypeStruct((BATCH, 128), jnp.uint32),
        scratch_types=[
            pltpu.VMEM((B_PER_W,), jnp.int32),
            pltpu.VMEM((B_PER_W, 128), jnp.uint32),
            pltpu.SemaphoreType.DMA,
        ],
    )
    def _sc_gather(row_hbm, t_hbm, out_hbm, idx_v, rows_v, sem):
        wid = lax.axis_index("s") * NC + lax.axis_index("c")
        base = wid * B_PER_W
        pltpu.sync_copy(row_hbm.at[pl.ds(base, B_PER_W)], idx_v)
        copies = []
        for t in range(NCH):
            sl = pl.ds(t * CHUNK, CHUNK)
            copies.append(pltpu.async_copy(t_hbm.at[idx_v.at[sl]], rows_v.at[sl], sem))
        for cp in copies:
            cp.wait()
        pltpu.sync_copy(rows_v, out_hbm.at[pl.ds(base, B_PER_W)])

    return _sc_gather


BT = 2048


def _mlp_body(u_ref, i_ref, qu_ref, qi_ref, ju_ref, ji_ref, c_ref,
              w1u4_ref, w1i4_ref, w1c_ref, b1_ref, w2_ref, b2_ref, o_ref):
    grp = lax.broadcasted_iota(jnp.int32, (BT, 128), 1) // EMBED

    def unpack(w_ref, j_ref):
        w = w_ref[...]
        bits = jnp.where(j_ref[...] == 0, w << 16, w & jnp.uint32(0xFFFF0000))
        return lax.bitcast_convert_type(bits, jnp.float32)

    um = jnp.where(grp == qu_ref[...], unpack(u_ref, ju_ref), 0.0)
    im = jnp.where(grp == qi_ref[...], unpack(i_ref, ji_ref), 0.0)
    h = (jnp.dot(um, w1u4_ref[...], preferred_element_type=jnp.float32)
         + jnp.dot(im, w1i4_ref[...], preferred_element_type=jnp.float32)
         + jnp.dot(c_ref[...], w1c_ref[...], preferred_element_type=jnp.float32)
         + b1_ref[...])
    h = jnp.maximum(h, 0.0)
    o_ref[...] = jnp.sum(h * w2_ref[...], axis=1, keepdims=True) + b2_ref[...]


def _mlp(u128, i128, qu, qi, ju, ji, content, w1u4, w1i4, w1c, b1, w2, b2):
    grid = (BATCH // BT,)
    return pl.pallas_call(
        _mlp_body,
        grid=grid,
        in_specs=[
            pl.BlockSpec((BT, 128), lambda i: (i, 0)),
            pl.BlockSpec((BT, 128), lambda i: (i, 0)),
            pl.BlockSpec((BT, 1), lambda i: (i, 0)),
            pl.BlockSpec((BT, 1), lambda i: (i, 0)),
            pl.BlockSpec((BT, 1), lambda i: (i, 0)),
            pl.BlockSpec((BT, 1), lambda i: (i, 0)),
            pl.BlockSpec((BT, CONTENT), lambda i: (i, 0)),
            pl.BlockSpec((128, HIDDEN), lambda i: (0, 0)),
            pl.BlockSpec((128, HIDDEN), lambda i: (0, 0)),
            pl.BlockSpec((CONTENT, HIDDEN), lambda i: (0, 0)),
            pl.BlockSpec((1, HIDDEN), lambda i: (0, 0)),
            pl.BlockSpec((1, HIDDEN), lambda i: (0, 0)),
            pl.BlockSpec((1, 1), lambda i: (0, 0)),
        ],
        out_specs=pl.BlockSpec((BT, 1), lambda i: (i, 0)),
        out_shape=jax.ShapeDtypeStruct((BATCH, 1), jnp.float32),
    )(u128, i128, qu, qi, ju, ji, content, w1u4, w1i4, w1c, b1, w2, b2)


def kernel(user_ids, item_ids, content_features, user_table, item_table,
           W1, b1, W2, b2):
    uids = user_ids.astype(jnp.int32)
    iids = item_ids.astype(jnp.int32)
    ur = uids // TP
    ir = iids // TP
    urow = (ur // NJQ) * TP + uids % TP
    irow = (ir // NJQ) * TP + iids % TP
    qu = (ur % 4).reshape(BATCH, 1)
    qi = (ir % 4).reshape(BATCH, 1)
    ju = ((ur // 4) % 2).reshape(BATCH, 1)
    ji = ((ir // 4) % 2).reshape(BATCH, 1)
    w1u = W1[:EMBED]
    w1i = W1[EMBED:2 * EMBED]
    w1c = W1[2 * EMBED:]
    eye = jnp.eye(128, dtype=jnp.float32)
    ut = _retile(user_table.T, eye)
    it = _retile(item_table.T, eye)
    gather = _make_sc_gather()
    u128 = gather(urow, ut)
    i128 = gather(irow, it)
    w1u4 = jnp.tile(w1u, (ROWS_PER_PAGE, 1))
    w1i4 = jnp.tile(w1i, (ROWS_PER_PAGE, 1))
    out = _mlp(u128, i128, qu, qi, ju, ji, content_features,
               w1u4, w1i4, w1c,
               b1.reshape(1, HIDDEN),
               W2.reshape(1, HIDDEN),
               b2.reshape(1, 1))
    return out


# speedup vs baseline: 4.9307x; 1.0723x over previous
import functools

import jax
import jax.numpy as jnp
from jax import lax
from jax.experimental import pallas as pl
from jax.experimental.pallas import tpu as pltpu
from jax.experimental.pallas import tpu_sc as plsc

BATCH = 16384
EMBED = 32
CONTENT = 64
HIDDEN = 64
NUSERS = 1000000
ROWS_PER_PAGE = 4

NC = 2
NS = 16
NW = NC * NS
B_PER_W = BATCH // NW
CHUNK = 128
NCH = B_PER_W // CHUNK


TP = 16384
NRUN = -(-NUSERS // TP)
NJQ = 8
TSTEPS = -(-NRUN // NJQ)
NPAGE = TSTEPS * TP


def _retile_body(*refs):
    t_refs, eye_ref, o_ref = refs[:NJQ], refs[NJQ], refs[NJQ + 1]
    halves = []
    for j in range(2):
        x = jnp.concatenate([t_refs[4 * j + q][...] for q in range(4)], axis=0)
        y = lax.dot_general(x, eye_ref[...], (((0,), (0,)), ((), ())),
                            preferred_element_type=jnp.float32)
        u16 = lax.bitcast_convert_type(y.astype(jnp.bfloat16), jnp.uint16)
        halves.append(u16.astype(jnp.uint32))
    o_ref[...] = halves[0] | (halves[1] << 16)


def _retile(table_t, eye):
    def spec(rq):
        return pl.BlockSpec(
            (EMBED, TP),
            lambda s, rq=rq: (0, jnp.minimum(NJQ * s + rq, NRUN - 1)))
    return pl.pallas_call(
        _retile_body,
        grid=(TSTEPS,),
        in_specs=[spec(rq) for rq in range(NJQ)]
        + [pl.BlockSpec((128, 128), lambda s: (0, 0))],
        out_specs=pl.BlockSpec((TP, 128), lambda s: (s, 0)),
        out_shape=jax.ShapeDtypeStruct((NPAGE, 128), jnp.uint32),
    )(*([table_t] * NJQ), eye)


@functools.lru_cache(maxsize=None)
def _make_sc_gather():
    mesh = plsc.VectorSubcoreMesh(core_axis_name="c", subcore_axis_name="s")

    @functools.partial(
        pl.kernel,
        mesh=mesh,
        out_type=jax.ShapeDtypeStruct((BATCH, 128), jnp.uint32),
        scratch_types=[
            pltpu.VMEM((B_PER_W,), jnp.int32),
            pltpu.VMEM((B_PER_W, 128), jnp.uint32),
            pltpu.SemaphoreType.DMA,
        ],
    )
    def _sc_gather(ids_hbm, t_hbm, out_hbm, idx_v, rows_v, sem):
        wid = lax.axis_index("s") * NC + lax.axis_index("c")
        base = wid * B_PER_W
        pltpu.sync_copy(ids_hbm.at[pl.ds(base, B_PER_W)], idx_v)
        for g in range(B_PER_W // 16):
            sl = pl.ds(g * 16, 16)
            v = idx_v[sl]
            idx_v[sl] = ((v >> 17) << 14) | (v & (TP - 1))
        copies = []
        for t in range(NCH):
            sl = pl.ds(t * CHUNK, CHUNK)
            copies.append(pltpu.async_copy(t_hbm.at[idx_v.at[sl]], rows_v.at[sl], sem))
        for cp in copies:
            cp.wait()
        pltpu.sync_copy(rows_v, out_hbm.at[pl.ds(base, B_PER_W)])

    return _sc_gather


BT = 2048


def _mlp_body(u_ref, i_ref, uid_ref, iid_ref, c_ref,
              w1u4_ref, w1i4_ref, w1c_ref, b1_ref, w2_ref, b2_ref, o_ref):
    grp = lax.broadcasted_iota(jnp.int32, (BT, 128), 1) // EMBED

    def unpack(w_ref, id_ref):
        w = w_ref[...]
        ids = id_ref[...]
        j = (ids >> 16) & 1
        q = (ids >> 14) & 3
        bits = jnp.where(j == 0, w << 16, w & jnp.uint32(0xFFFF0000))
        return lax.bitcast_convert_type(bits, jnp.float32), q

    uval, qu = unpack(u_ref, uid_ref)
    ival, qi = unpack(i_ref, iid_ref)
    um = jnp.where(grp == qu, uval, 0.0)
    im = jnp.where(grp == qi, ival, 0.0)
    h = (jnp.dot(um, w1u4_ref[...], preferred_element_type=jnp.float32)
         + jnp.dot(im, w1i4_ref[...], preferred_element_type=jnp.float32)
         + jnp.dot(c_ref[...], w1c_ref[...], preferred_element_type=jnp.float32)
         + b1_ref[...])
    h = jnp.maximum(h, 0.0)
    o_ref[...] = jnp.sum(h * w2_ref[...], axis=1, keepdims=True) + b2_ref[...]


def _mlp(u128, i128, uidc, iidc, content, w1u4, w1i4, w1c, b1, w2, b2):
    grid = (BATCH // BT,)
    return pl.pallas_call(
        _mlp_body,
        grid=grid,
        in_specs=[
            pl.BlockSpec((BT, 128), lambda i: (i, 0)),
            pl.BlockSpec((BT, 128), lambda i: (i, 0)),
            pl.BlockSpec((BT, 1), lambda i: (i, 0)),
            pl.BlockSpec((BT, 1), lambda i: (i, 0)),
            pl.BlockSpec((BT, CONTENT), lambda i: (i, 0)),
            pl.BlockSpec((128, HIDDEN), lambda i: (0, 0)),
            pl.BlockSpec((128, HIDDEN), lambda i: (0, 0)),
            pl.BlockSpec((CONTENT, HIDDEN), lambda i: (0, 0)),
            pl.BlockSpec((1, HIDDEN), lambda i: (0, 0)),
            pl.BlockSpec((1, HIDDEN), lambda i: (0, 0)),
            pl.BlockSpec((1, 1), lambda i: (0, 0)),
        ],
        out_specs=pl.BlockSpec((BT, 1), lambda i: (i, 0)),
        out_shape=jax.ShapeDtypeStruct((BATCH, 1), jnp.float32),
    )(u128, i128, uidc, iidc, content, w1u4, w1i4, w1c, b1, w2, b2)


def kernel(user_ids, item_ids, content_features, user_table, item_table,
           W1, b1, W2, b2):
    uids = user_ids.astype(jnp.int32)
    iids = item_ids.astype(jnp.int32)
    w1u = W1[:EMBED]
    w1i = W1[EMBED:2 * EMBED]
    w1c = W1[2 * EMBED:]
    eye = jnp.eye(128, dtype=jnp.float32)
    ut = _retile(user_table.T, eye)
    it = _retile(item_table.T, eye)
    gather = _make_sc_gather()
    u128 = gather(uids, ut)
    i128 = gather(iids, it)
    w1u4 = jnp.tile(w1u, (ROWS_PER_PAGE, 1))
    w1i4 = jnp.tile(w1i, (ROWS_PER_PAGE, 1))
    out = _mlp(u128, i128, uids.reshape(BATCH, 1), iids.reshape(BATCH, 1),
               content_features,
               w1u4, w1i4, w1c,
               b1.reshape(1, HIDDEN),
               W2.reshape(1, HIDDEN),
               b2.reshape(1, 1))
    return out


# speedup vs baseline: 4.9462x; 1.0031x over previous
import functools

import jax
import jax.numpy as jnp
from jax import lax
from jax.experimental import pallas as pl
from jax.experimental.pallas import tpu as pltpu
from jax.experimental.pallas import tpu_sc as plsc

BATCH = 16384
EMBED = 32
CONTENT = 64
HIDDEN = 64
NUSERS = 1000000
ROWS_PER_PAGE = 4

NC = 2
NS = 16
NW = NC * NS
B_PER_W = BATCH // NW
CHUNK = 128
NCH = B_PER_W // CHUNK


TP = 16384
NRUN = -(-NUSERS // TP)
NJQ = 8
TSTEPS = -(-NRUN // NJQ)
NPAGE = TSTEPS * TP


def _retile_body(*refs):
    t_refs, eye_ref, o_ref = refs[:NJQ], refs[NJQ], refs[NJQ + 1]
    halves = []
    for j in range(2):
        x = jnp.concatenate([t_refs[4 * j + q][...] for q in range(4)], axis=0)
        y = lax.dot_general(x, eye_ref[...], (((0,), (0,)), ((), ())),
                            preferred_element_type=jnp.float32)
        u16 = lax.bitcast_convert_type(y.astype(jnp.bfloat16), jnp.uint16)
        halves.append(u16.astype(jnp.uint32))
    o_ref[...] = halves[0] | (halves[1] << 16)


def _retile(table_t, eye):
    def spec(rq):
        return pl.BlockSpec(
            (EMBED, TP),
            lambda s, rq=rq: (0, jnp.minimum(NJQ * s + rq, NRUN - 1)))
    return pl.pallas_call(
        _retile_body,
        grid=(TSTEPS,),
        in_specs=[spec(rq) for rq in range(NJQ)]
        + [pl.BlockSpec((128, 128), lambda s: (0, 0))],
        out_specs=pl.BlockSpec((TP, 128), lambda s: (s, 0)),
        out_shape=jax.ShapeDtypeStruct((NPAGE, 128), jnp.uint32),
    )(*([table_t] * NJQ), eye)


@functools.lru_cache(maxsize=None)
def _make_sc_gather():
    mesh = plsc.VectorSubcoreMesh(core_axis_name="c", subcore_axis_name="s")

    @functools.partial(
        pl.kernel,
        mesh=mesh,
        out_type=jax.ShapeDtypeStruct((BATCH, 128), jnp.uint32),
        scratch_types=[
            pltpu.VMEM((B_PER_W,), jnp.int32),
            pltpu.VMEM((B_PER_W, 128), jnp.uint32),
            pltpu.SemaphoreType.DMA,
        ],
    )
    def _sc_gather(ids_hbm, t_hbm, out_hbm, idx_v, rows_v, sem):
        wid = lax.axis_index("s") * NC + lax.axis_index("c")
        base = wid * B_PER_W
        pltpu.sync_copy(ids_hbm.at[pl.ds(base, B_PER_W)], idx_v)
        for g in range(B_PER_W // 16):
            sl = pl.ds(g * 16, 16)
            v = idx_v[sl]
            idx_v[sl] = ((v >> 17) << 14) | (v & (TP - 1))
        copies = []
        for t in range(NCH):
            sl = pl.ds(t * CHUNK, CHUNK)
            copies.append(pltpu.async_copy(t_hbm.at[idx_v.at[sl]], rows_v.at[sl], sem))
        for cp in copies:
            cp.wait()
        pltpu.sync_copy(rows_v, out_hbm.at[pl.ds(base, B_PER_W)])

    return _sc_gather


BT = 4096


def _mlp_body(u_ref, i_ref, uid_ref, iid_ref, c_ref,
              w1u4_ref, w1i4_ref, w1c_ref, b1_ref, w2_ref, b2_ref, o_ref):
    grp = lax.broadcasted_iota(jnp.int32, (BT, 128), 1) // EMBED

    def unpack(w_ref, id_ref):
        w = w_ref[...]
        ids = id_ref[...]
        j = (ids >> 16) & 1
        q = (ids >> 14) & 3
        bits = jnp.where(j == 0, w << 16, w & jnp.uint32(0xFFFF0000))
        return lax.bitcast_convert_type(bits, jnp.float32), q

    uval, qu = unpack(u_ref, uid_ref)
    ival, qi = unpack(i_ref, iid_ref)
    um = jnp.where(grp == qu, uval, 0.0)
    im = jnp.where(grp == qi, ival, 0.0)
    h = (jnp.dot(um, w1u4_ref[...], preferred_element_type=jnp.float32)
         + jnp.dot(im, w1i4_ref[...], preferred_element_type=jnp.float32)
         + jnp.dot(c_ref[...], w1c_ref[...], preferred_element_type=jnp.float32)
         + b1_ref[...])
    h = jnp.maximum(h, 0.0)
    o_ref[...] = jnp.sum(h * w2_ref[...], axis=1, keepdims=True) + b2_ref[...]


def _mlp(u128, i128, uidc, iidc, content, w1u4, w1i4, w1c, b1, w2, b2):
    grid = (BATCH // BT,)
    return pl.pallas_call(
        _mlp_body,
        grid=grid,
        in_specs=[
            pl.BlockSpec((BT, 128), lambda i: (i, 0)),
            pl.BlockSpec((BT, 128), lambda i: (i, 0)),
            pl.BlockSpec((BT, 1), lambda i: (i, 0)),
            pl.BlockSpec((BT, 1), lambda i: (i, 0)),
            pl.BlockSpec((BT, CONTENT), lambda i: (i, 0)),
            pl.BlockSpec((128, HIDDEN), lambda i: (0, 0)),
            pl.BlockSpec((128, HIDDEN), lambda i: (0, 0)),
            pl.BlockSpec((CONTENT, HIDDEN), lambda i: (0, 0)),
            pl.BlockSpec((1, HIDDEN), lambda i: (0, 0)),
            pl.BlockSpec((1, HIDDEN), lambda i: (0, 0)),
            pl.BlockSpec((1, 1), lambda i: (0, 0)),
        ],
        out_specs=pl.BlockSpec((BT, 1), lambda i: (i, 0)),
        out_shape=jax.ShapeDtypeStruct((BATCH, 1), jnp.float32),
    )(u128, i128, uidc, iidc, content, w1u4, w1i4, w1c, b1, w2, b2)


def kernel(user_ids, item_ids, content_features, user_table, item_table,
           W1, b1, W2, b2):
    uids = user_ids.astype(jnp.int32)
    iids = item_ids.astype(jnp.int32)
    w1u = W1[:EMBED]
    w1i = W1[EMBED:2 * EMBED]
    w1c = W1[2 * EMBED:]
    eye = jnp.eye(128, dtype=jnp.float32)
    gather = _make_sc_gather()
    ut = _retile(user_table.T, eye)
    u128 = gather(uids, ut)
    it = _retile(item_table.T, eye)
    i128 = gather(iids, it)
    w1u4 = jnp.tile(w1u, (ROWS_PER_PAGE, 1))
    w1i4 = jnp.tile(w1i, (ROWS_PER_PAGE, 1))
    out = _mlp(u128, i128, uids.reshape(BATCH, 1), iids.reshape(BATCH, 1),
               content_features,
               w1u4, w1i4, w1c,
               b1.reshape(1, HIDDEN),
               W2.reshape(1, HIDDEN),
               b2.reshape(1, 1))
    return out


# speedup vs baseline: 4.9520x; 1.0012x over previous
import functools

import jax
import jax.numpy as jnp
from jax import lax
from jax.experimental import pallas as pl
from jax.experimental.pallas import tpu as pltpu
from jax.experimental.pallas import tpu_sc as plsc

BATCH = 16384
EMBED = 32
CONTENT = 64
HIDDEN = 64
NUSERS = 1000000
ROWS_PER_PAGE = 4

NC = 2
NS = 16
NW = NC * NS
B_PER_W = BATCH // NW
CHUNK = 128
NCH = B_PER_W // CHUNK


TP = 16384
LOG_TP = TP.bit_length() - 1
NRUN = -(-NUSERS // TP)
NJQ = 8
TSTEPS = -(-NRUN // NJQ)
NPAGE = TSTEPS * TP


def _retile_body(*refs):
    t_refs, eye_ref, o_ref = refs[:NJQ], refs[NJQ], refs[NJQ + 1]
    halves = []
    for j in range(2):
        x = jnp.concatenate([t_refs[4 * j + q][...] for q in range(4)], axis=0)
        y = lax.dot_general(x, eye_ref[...], (((0,), (0,)), ((), ())),
                            preferred_element_type=jnp.float32)
        u16 = lax.bitcast_convert_type(y.astype(jnp.bfloat16), jnp.uint16)
        halves.append(u16.astype(jnp.uint32))
    o_ref[...] = halves[0] | (halves[1] << 16)


def _retile(table_t, eye):
    def spec(rq):
        return pl.BlockSpec(
            (EMBED, TP),
            lambda s, rq=rq: (0, jnp.minimum(NJQ * s + rq, NRUN - 1)))
    return pl.pallas_call(
        _retile_body,
        grid=(TSTEPS,),
        in_specs=[spec(rq) for rq in range(NJQ)]
        + [pl.BlockSpec((128, 128), lambda s: (0, 0))],
        out_specs=pl.BlockSpec((TP, 128), lambda s: (s, 0)),
        out_shape=jax.ShapeDtypeStruct((NPAGE, 128), jnp.uint32),
    )(*([table_t] * NJQ), eye)


@functools.lru_cache(maxsize=None)
def _make_sc_gather():
    mesh = plsc.VectorSubcoreMesh(core_axis_name="c", subcore_axis_name="s")

    @functools.partial(
        pl.kernel,
        mesh=mesh,
        out_type=jax.ShapeDtypeStruct((BATCH, 128), jnp.uint32),
        scratch_types=[
            pltpu.VMEM((B_PER_W,), jnp.int32),
            pltpu.VMEM((B_PER_W, 128), jnp.uint32),
            pltpu.SemaphoreType.DMA,
        ],
    )
    def _sc_gather(ids_hbm, t_hbm, out_hbm, idx_v, rows_v, sem):
        wid = lax.axis_index("s") * NC + lax.axis_index("c")
        base = wid * B_PER_W
        pltpu.sync_copy(ids_hbm.at[pl.ds(base, B_PER_W)], idx_v)
        for g in range(B_PER_W // 16):
            sl = pl.ds(g * 16, 16)
            v = idx_v[sl]
            idx_v[sl] = ((v >> (LOG_TP + 3)) << LOG_TP) | (v & (TP - 1))
        copies = []
        for t in range(NCH):
            sl = pl.ds(t * CHUNK, CHUNK)
            copies.append(pltpu.async_copy(t_hbm.at[idx_v.at[sl]], rows_v.at[sl], sem))
        for cp in copies:
            cp.wait()
        pltpu.sync_copy(rows_v, out_hbm.at[pl.ds(base, B_PER_W)])

    return _sc_gather


BT = 4096


def _mlp_body(u_ref, i_ref, uid_ref, iid_ref, c_ref,
              w1u4_ref, w1i4_ref, w1c_ref, b1_ref, w2_ref, b2_ref, o_ref):
    grp = lax.broadcasted_iota(jnp.int32, (BT, 128), 1) // EMBED

    def unpack(w_ref, id_ref):
        w = w_ref[...]
        ids = id_ref[...]
        j = (ids >> (LOG_TP + 2)) & 1
        q = (ids >> LOG_TP) & 3
        bits = jnp.where(j == 0, w << 16, w & jnp.uint32(0xFFFF0000))
        return lax.bitcast_convert_type(bits, jnp.float32), q

    uval, qu = unpack(u_ref, uid_ref)
    ival, qi = unpack(i_ref, iid_ref)
    um = jnp.where(grp == qu, uval, 0.0)
    im = jnp.where(grp == qi, ival, 0.0)
    h = (jnp.dot(um, w1u4_ref[...], preferred_element_type=jnp.float32)
         + jnp.dot(im, w1i4_ref[...], preferred_element_type=jnp.float32)
         + jnp.dot(c_ref[...], w1c_ref[...], preferred_element_type=jnp.float32)
         + b1_ref[...])
    h = jnp.maximum(h, 0.0)
    o_ref[...] = jnp.sum(h * w2_ref[...], axis=1, keepdims=True) + b2_ref[...]


def _mlp(u128, i128, uidc, iidc, content, w1u4, w1i4, w1c, b1, w2, b2):
    grid = (BATCH // BT,)
    return pl.pallas_call(
        _mlp_body,
        grid=grid,
        in_specs=[
            pl.BlockSpec((BT, 128), lambda i: (i, 0)),
            pl.BlockSpec((BT, 128), lambda i: (i, 0)),
            pl.BlockSpec((BT, 1), lambda i: (i, 0)),
            pl.BlockSpec((BT, 1), lambda i: (i, 0)),
            pl.BlockSpec((BT, CONTENT), lambda i: (i, 0)),
            pl.BlockSpec((128, HIDDEN), lambda i: (0, 0)),
            pl.BlockSpec((128, HIDDEN), lambda i: (0, 0)),
            pl.BlockSpec((CONTENT, HIDDEN), lambda i: (0, 0)),
            pl.BlockSpec((1, HIDDEN), lambda i: (0, 0)),
            pl.BlockSpec((1, HIDDEN), lambda i: (0, 0)),
            pl.BlockSpec((1, 1), lambda i: (0, 0)),
        ],
        out_specs=pl.BlockSpec((BT, 1), lambda i: (i, 0)),
        out_shape=jax.ShapeDtypeStruct((BATCH, 1), jnp.float32),
    )(u128, i128, uidc, iidc, content, w1u4, w1i4, w1c, b1, w2, b2)


def kernel(user_ids, item_ids, content_features, user_table, item_table,
           W1, b1, W2, b2):
    uids = user_ids.astype(jnp.int32)
    iids = item_ids.astype(jnp.int32)
    w1u = W1[:EMBED]
    w1i = W1[EMBED:2 * EMBED]
    w1c = W1[2 * EMBED:]
    eye = jnp.eye(128, dtype=jnp.float32)
    gather = _make_sc_gather()
    ut = _retile(user_table.T, eye)
    u128 = gather(uids, ut)
    it = _retile(item_table.T, eye)
    i128 = gather(iids, it)
    w1u4 = jnp.tile(w1u, (ROWS_PER_PAGE, 1))
    w1i4 = jnp.tile(w1i, (ROWS_PER_PAGE, 1))
    out = _mlp(u128, i128, uids.reshape(BATCH, 1), iids.reshape(BATCH, 1),
               content_features,
               w1u4, w1i4, w1c,
               b1.reshape(1, HIDDEN),
               W2.reshape(1, HIDDEN),
               b2.reshape(1, 1))
    return out
